# baseline (speedup 1.0000x reference)
import functools

import jax
import jax.numpy as jnp
from jax import lax
from jax.experimental import pallas as pl
from jax.experimental.pallas import tpu as pltpu
from jax.experimental.pallas import tpu_sc as plsc

_HI = lax.Precision.HIGHEST

_B, _N, _INF, _TF, _D, _K = 2, 1024, 64, 128, 256, 16
_TQ = 256
_TM = 128
_TW = 256
_TA = 512
_BF = jnp.bfloat16


def _bmm(a, b_ref):
    return jnp.dot(a.astype(_BF), b_ref[:], preferred_element_type=jnp.float32)


def _wprep_body(W1, b1, Wq, Wk, Wv, Wg1, Wd2, bd2, bg1,
                Aq, cq, Ak, ck, Av, cv, Wd2g):
    w1 = W1[:]
    g1 = Wg1[:]
    Aq[:] = jnp.dot(jnp.dot(w1, Wq[:], precision=_HI), g1, precision=_HI)
    Ak[:] = jnp.dot(jnp.dot(w1, Wk[:], precision=_HI), g1, precision=_HI)
    Av[:] = jnp.dot(w1, Wv[:], precision=_HI)
    b1v = b1[:]
    cq[:] = (jnp.dot(jnp.dot(b1v, Wq[:], precision=_HI), g1, precision=_HI)
             + jnp.dot(bd2[:], g1, precision=_HI) + bg1[:])
    ck[:] = jnp.dot(jnp.dot(b1v, Wk[:], precision=_HI), g1, precision=_HI)
    cv[:] = jnp.dot(b1v, Wv[:], precision=_HI)
    Wd2g[:] = jnp.dot(Wd2[:], g1, precision=_HI)


def _wprep(W1, b1, Wq, Wk, Wv, Wg1, Wd2, bd2, bg1):
    f32 = jnp.float32
    outs = (
        jax.ShapeDtypeStruct((_TF, _D), f32),
        jax.ShapeDtypeStruct((1, _D), f32),
        jax.ShapeDtypeStruct((_TF, _D), f32),
        jax.ShapeDtypeStruct((1, _D), f32),
        jax.ShapeDtypeStruct((_TF, _D), f32),
        jax.ShapeDtypeStruct((1, _D), f32),
        jax.ShapeDtypeStruct((_D, _D), f32),
    )
    return pl.pallas_call(_wprep_body, out_shape=outs)(
        W1, b1.reshape(1, _D), Wq, Wk, Wv, Wg1, Wd2,
        bd2.reshape(1, _D), bg1.reshape(1, _D))


def _feats_body(xq_ref, W0a, b0a, W0b, b0b, table_ref):
    xq = xq_ref[:]
    f1 = jnp.maximum(jnp.dot(xq, W0a[:]) + b0a[:], 0.0)
    feats = jnp.dot(f1, W0b[:]) + b0b[:]
    table_ref[:] = jnp.concatenate(
        [xq, jnp.zeros((_TA, _INF), jnp.float32), feats], axis=1)


def _feats(xyzf, W0a, b0a, W0b, b0b):
    grid = (_B * _N // _TA,)
    row = lambda t: (t, 0)
    full = lambda t: (0, 0)
    return pl.pallas_call(
        _feats_body, grid=grid,
        in_specs=[
            pl.BlockSpec((_TA, _INF), row),
            pl.BlockSpec((_INF, _TF), full),
            pl.BlockSpec((1, _TF), full),
            pl.BlockSpec((_TF, _TF), full),
            pl.BlockSpec((1, _TF), full),
        ],
        out_specs=pl.BlockSpec((_TA, _TW), row),
        out_shape=jax.ShapeDtypeStruct((_B * _N, _TW), jnp.float32),
    )(xyzf, W0a, b0a.reshape(1, _TF), W0b, b0b.reshape(1, _TF))


def _topk_body(xq_ref, xf_ref, idx_ref, *, batch):
    xq = xq_ref[:]
    xf = xf_ref[:]

    d = -2.0 * lax.dot_general(xq, xf, (((1,), (1,)), ((), ())))
    d = d + jnp.sum(xq * xq, axis=1, keepdims=True)
    d = d + jnp.sum(xf * xf, axis=1)[None, :]

    colf = lax.broadcasted_iota(jnp.int32, (_TQ, _N), 1).astype(jnp.float32)
    big = jnp.float32(3.0e38)
    vals = d
    sels = []
    for _ in range(_K):
        m = jnp.min(vals, axis=1, keepdims=True)
        cand = jnp.where(vals <= m, colf, jnp.float32(_N))
        sel = jnp.min(cand, axis=1, keepdims=True)
        sels.append(sel)
        vals = jnp.where(colf == sel, big, vals)
    idx_ref[:] = jnp.concatenate(sels, axis=1).astype(jnp.int32) + batch * _N


def _topk(batch, xyzf):
    nt = _N // _TQ
    grid = (nt,)
    return pl.pallas_call(
        functools.partial(_topk_body, batch=batch),
        grid=grid,
        in_specs=[
            pl.BlockSpec((_TQ, _INF), lambda t: (batch * nt + t, 0)),
            pl.BlockSpec((_N, _INF), lambda t: (batch, 0)),
        ],
        out_specs=pl.BlockSpec((_TQ, _K), lambda t: (t, 0)),
        out_shape=jax.ShapeDtypeStruct((_N, _K), jnp.int32),
    )(xyzf, xyzf)


def _sc_gather(table, idx_flat):
    tot = idx_flat.shape[0]
    nw = 32
    per_w = tot // nw
    ch = 128
    n_ch = per_w // ch

    mesh = plsc.VectorSubcoreMesh(core_axis_name="c", subcore_axis_name="s")

    @functools.partial(
        pl.kernel, mesh=mesh,
        out_type=jax.ShapeDtypeStruct((tot, _TW), jnp.float32),
        scratch_types=[
            pltpu.VMEM((per_w,), jnp.int32),
            pltpu.VMEM((ch, _TW), jnp.float32),
            pltpu.VMEM((ch, _TW), jnp.float32),
            pltpu.SemaphoreType.DMA,
            pltpu.SemaphoreType.DMA,
        ],
    )
    def gather_kernel(table_hbm, idx_hbm, out_hbm, idx_v, rows_a, rows_b,
                      sem_a, sem_b):
        wid = lax.axis_index("s") * 2 + lax.axis_index("c")
        base = wid * per_w
        pltpu.sync_copy(idx_hbm.at[pl.ds(base, per_w)], idx_v)
        bufs = (rows_a, rows_b)
        sems = (sem_a, sem_b)
        cps = [None, None]
        for c in range(n_ch + 1):
            if c < n_ch:
                p = c % 2
                cps[p] = pltpu.async_copy(
                    table_hbm.at[idx_v.at[pl.ds(c * ch, ch)]],
                    bufs[p], sems[p])
            if c >= 1:
                p = (c - 1) % 2
                cps[p].wait()
                pltpu.sync_copy(bufs[p],
                                out_hbm.at[pl.ds(base + (c - 1) * ch, ch)])

    return gather_kernel(table, idx_flat)


def _stage3_body(g_ref, xyz_ref, pre_ref, E_ref,
                 Aqb, cq, Akvb, ckv,
                 Wd1b, bd1, Wd2cb, bd2, Wg2b, bg2, W2, b2,
                 attn_ref, res_ref):
    g = g_ref[:]
    xq = xyz_ref[:]
    fq = pre_ref[:]
    pos = xq[:, None, :] - g[:, :, 0:_INF]
    pos2 = pos.reshape(_TM * _K, _INF)
    h = jnp.maximum(_bmm(pos2, Wd1b) + bd1[:], 0.0)
    hb = h.astype(_BF)
    hw = jnp.dot(hb, Wd2cb[:], preferred_element_type=jnp.float32)
    pe = hw[:, 0:_D] + bd2[:]
    a3 = hw[:, _D:]

    f2 = g[:, :, _TF:].reshape(_TM * _K, _TF)
    kv = _bmm(f2, Akvb) + ckv[:]
    kg2 = kv[:, 0:_D]
    v2 = kv[:, _D:]
    qg = _bmm(fq, Aqb) + cq[:]
    qg2 = jnp.broadcast_to(qg[:, None, :], (_TM, _K, _D))
    qg2 = qg2.reshape(_TM * _K, _D)

    l1 = jnp.maximum(qg2 - kg2 + a3, 0.0)
    logits = (_bmm(l1, Wg2b) + bg2[:]) * jnp.float32(1.0 / 16.0)

    lg3 = logits.reshape(_TM, _K, _D)
    m = jnp.max(lg3, axis=1, keepdims=True)
    e = jnp.exp(lg3 - m)
    s = jnp.sum(e, axis=1, keepdims=True)
    attn = e / s
    attn_ref[:] = attn

    pe3 = pe.reshape(_TM, _K, _D)
    v3 = v2.reshape(_TM, _K, _D)
    p2 = (attn * (v3 + pe3)).reshape(_TM * _K, _D)
    wsum = jnp.dot(E_ref[:], p2, preferred_element_type=jnp.float32)
    res_ref[:] = jnp.dot(wsum, W2[:]) + b2[:] + fq


def _stage3_body_alias(attn_in, res_in, *args):
    _stage3_body(*args)


def _stage3(batch, g3, xyzf, table, E, wp, prev):
    f32 = jnp.float32
    (Aqb, cq, Akvb, ckv, Wd1b, Wd2cb, Wg2b) = wp[0]
    (bd1, bd2, bg2, W2, b2) = wp[1]
    nt = _N // _TM
    grid = (nt,)
    row = lambda t: (batch * nt + t, 0)
    row3 = lambda t: (batch * nt + t, 0, 0)
    grow = lambda t: (t, 0, 0)
    row_f = lambda t: (batch * nt + t, 1)
    full = lambda t: (0, 0)
    in_specs = [
        pl.BlockSpec((_TM, _K, _TW), grow),
        pl.BlockSpec((_TM, _INF), row),
        pl.BlockSpec((_TM, _TF), row_f),
        pl.BlockSpec((_TM, _TM * _K), full),
        pl.BlockSpec((_TF, _D), full),
        pl.BlockSpec((1, _D), full),
        pl.BlockSpec((_TF, 2 * _D), full),
        pl.BlockSpec((1, 2 * _D), full),
        pl.BlockSpec((_INF, _D), full),
        pl.BlockSpec((1, _D), full),
        pl.BlockSpec((_D, 2 * _D), full),
        pl.BlockSpec((1, _D), full),
        pl.BlockSpec((_D, _D), full),
        pl.BlockSpec((1, _D), full),
        pl.BlockSpec((_D, _TF), full),
        pl.BlockSpec((1, _TF), full),
    ]
    out_specs = [
        pl.BlockSpec((_TM, _K, _D), row3),
        pl.BlockSpec((_TM, _TF), row),
    ]
    outs = (
        jax.ShapeDtypeStruct((_B * _N, _K, _D), f32),
        jax.ShapeDtypeStruct((_B * _N, _TF), f32),
    )
    operands = (g3, xyzf, table, E, Aqb, cq, Akvb, ckv,
                Wd1b, bd1.reshape(1, _D), Wd2cb, bd2.reshape(1, _D),
                Wg2b, bg2.reshape(1, _D), W2, b2.reshape(1, _TF))
    if prev is None:
        return pl.pallas_call(
            _stage3_body, grid=grid, in_specs=in_specs,
            out_specs=out_specs, out_shape=outs)(*operands)
    attn_prev, res_prev = prev
    in_specs = [pl.BlockSpec(memory_space=pl.ANY),
                pl.BlockSpec(memory_space=pl.ANY)] + in_specs
    return pl.pallas_call(
        _stage3_body_alias, grid=grid, in_specs=in_specs,
        out_specs=out_specs, out_shape=outs,
        input_output_aliases={0: 0, 1: 1},
    )(attn_prev, res_prev, *operands)


def kernel(xyz, W0a, b0a, W0b, b0b, W1, b1, W2, b2, Wd1, bd1, Wd2, bd2,
           Wg1, bg1, Wg2, bg2, Wq, Wk, Wv):
    assert xyz.shape == (_B, _N, _INF)
    Aq, cq, Ak, ck, Av, cv, Wd2g = _wprep(W1, b1, Wq, Wk, Wv, Wg1, Wd2,
                                          bd2, bg1)
    Akvb = jnp.concatenate([Ak, Av], axis=1).astype(_BF)
    ckv = jnp.concatenate([ck.reshape(1, _D), cv.reshape(1, _D)], axis=1)
    Wd2cb = jnp.concatenate([Wd2, Wd2g], axis=1).astype(_BF)
    wp = ((Aq.astype(_BF), cq.reshape(1, _D), Akvb, ckv,
           Wd1.astype(_BF), Wd2cb, Wg2.astype(_BF)),
          (bd1, bd2, bg2, W2, b2))
    rows = jnp.arange(_TM, dtype=jnp.int32)[:, None]
    cols = jnp.arange(_TM * _K, dtype=jnp.int32)[None, :]
    E = (cols // _K == rows).astype(jnp.float32)
    xyzf = xyz.reshape(_B * _N, _INF)
    table = _feats(xyzf, W0a, b0a, W0b, b0b)
    prev = None
    for b in range(_B):
        idx = _topk(b, xyzf)
        g = _sc_gather(table, idx.reshape(_N * _K))
        prev = _stage3(b, g.reshape(_N, _K, _TW), xyzf, table, E, wp, prev)
    attn, res = prev[0], prev[1]
    return (res.reshape(_B, _N, _TF), attn.reshape(_B, _N, _K, _D))


# speedup vs baseline: 1.0140x; 1.0140x over previous
import functools

import jax
import jax.numpy as jnp
from jax import lax
from jax.experimental import pallas as pl
from jax.experimental.pallas import tpu as pltpu
from jax.experimental.pallas import tpu_sc as plsc

_HI = lax.Precision.HIGHEST

_B, _N, _INF, _TF, _D, _K = 2, 1024, 64, 128, 256, 16
_TQ = 256
_TM = 128
_TW = 256
_TA = 512
_BF = jnp.bfloat16


def _bmm(a, b_ref):
    return jnp.dot(a.astype(_BF), b_ref[:], preferred_element_type=jnp.float32)


def _wprep_body(W1, b1, Wq, Wk, Wv, Wg1, Wd2, bd2, bg1, Wd1, Wg2,
                Aqb, cq, Akvb, ckv, Wd2cb, Wd1b, Wg2b):
    Wd1b[:] = Wd1[:].astype(_BF)
    Wg2b[:] = Wg2[:].astype(_BF)
    w1 = W1[:]
    g1 = Wg1[:]
    Aq = jnp.dot(jnp.dot(w1, Wq[:], precision=_HI), g1, precision=_HI)
    Ak = jnp.dot(jnp.dot(w1, Wk[:], precision=_HI), g1, precision=_HI)
    Av = jnp.dot(w1, Wv[:], precision=_HI)
    Aqb[:] = Aq.astype(_BF)
    Akvb[:] = jnp.concatenate([Ak, Av], axis=1).astype(_BF)
    b1v = b1[:]
    cq[:] = (jnp.dot(jnp.dot(b1v, Wq[:], precision=_HI), g1, precision=_HI)
             + jnp.dot(bd2[:], g1, precision=_HI) + bg1[:])
    ck = jnp.dot(jnp.dot(b1v, Wk[:], precision=_HI), g1, precision=_HI)
    cv = jnp.dot(b1v, Wv[:], precision=_HI)
    ckv[:] = jnp.concatenate([ck, cv], axis=1)
    Wd2g = jnp.dot(Wd2[:], g1, precision=_HI)
    Wd2cb[:] = jnp.concatenate([Wd2[:], Wd2g], axis=1).astype(_BF)


def _wprep(W1, b1, Wq, Wk, Wv, Wg1, Wd2, bd2, bg1, Wd1, Wg2):
    f32 = jnp.float32
    outs = (
        jax.ShapeDtypeStruct((_TF, _D), _BF),
        jax.ShapeDtypeStruct((1, _D), f32),
        jax.ShapeDtypeStruct((_TF, 2 * _D), _BF),
        jax.ShapeDtypeStruct((1, 2 * _D), f32),
        jax.ShapeDtypeStruct((_D, 2 * _D), _BF),
        jax.ShapeDtypeStruct((_INF, _D), _BF),
        jax.ShapeDtypeStruct((_D, _D), _BF),
    )
    return pl.pallas_call(_wprep_body, out_shape=outs)(
        W1, b1.reshape(1, _D), Wq, Wk, Wv, Wg1, Wd2,
        bd2.reshape(1, _D), bg1.reshape(1, _D), Wd1, Wg2)


def _feats_body(xq_ref, W0a, b0a, W0b, b0b, table_ref):
    xq = xq_ref[:]
    f1 = jnp.maximum(jnp.dot(xq, W0a[:]) + b0a[:], 0.0)
    feats = jnp.dot(f1, W0b[:]) + b0b[:]
    table_ref[:] = jnp.concatenate(
        [xq, jnp.zeros((_TA, _INF), jnp.float32), feats], axis=1)


def _feats(xyzf, W0a, b0a, W0b, b0b):
    grid = (_B * _N // _TA,)
    row = lambda t: (t, 0)
    full = lambda t: (0, 0)
    return pl.pallas_call(
        _feats_body, grid=grid,
        in_specs=[
            pl.BlockSpec((_TA, _INF), row),
            pl.BlockSpec((_INF, _TF), full),
            pl.BlockSpec((1, _TF), full),
            pl.BlockSpec((_TF, _TF), full),
            pl.BlockSpec((1, _TF), full),
        ],
        out_specs=pl.BlockSpec((_TA, _TW), row),
        out_shape=jax.ShapeDtypeStruct((_B * _N, _TW), jnp.float32),
    )(xyzf, W0a, b0a.reshape(1, _TF), W0b, b0b.reshape(1, _TF))


def _topk_body(xq_ref, xf_ref, idx_ref, *, batch):
    xq = xq_ref[:]
    xf = xf_ref[:]

    d = -2.0 * lax.dot_general(xq, xf, (((1,), (1,)), ((), ())))
    d = d + jnp.sum(xq * xq, axis=1, keepdims=True)
    d = d + jnp.sum(xf * xf, axis=1)[None, :]

    colf = lax.broadcasted_iota(jnp.int32, (_TQ, _N), 1).astype(jnp.float32)
    big = jnp.float32(3.0e38)
    vals = d
    sels = []
    for _ in range(_K):
        m = jnp.min(vals, axis=1, keepdims=True)
        cand = jnp.where(vals <= m, colf, jnp.float32(_N))
        sel = jnp.min(cand, axis=1, keepdims=True)
        sels.append(sel)
        vals = jnp.where(colf == sel, big, vals)
    idx_ref[:] = jnp.concatenate(sels, axis=1).astype(jnp.int32) + batch * _N


def _topk(batch, xyzf):
    nt = _N // _TQ
    grid = (nt,)
    return pl.pallas_call(
        functools.partial(_topk_body, batch=batch),
        grid=grid,
        in_specs=[
            pl.BlockSpec((_TQ, _INF), lambda t: (batch * nt + t, 0)),
            pl.BlockSpec((_N, _INF), lambda t: (batch, 0)),
        ],
        out_specs=pl.BlockSpec((_TQ, _K), lambda t: (t, 0)),
        out_shape=jax.ShapeDtypeStruct((_N, _K), jnp.int32),
    )(xyzf, xyzf)


def _sc_gather(table, idx_flat):
    tot = idx_flat.shape[0]
    nw = 32
    per_w = tot // nw
    ch = 128
    n_ch = per_w // ch

    mesh = plsc.VectorSubcoreMesh(core_axis_name="c", subcore_axis_name="s")

    @functools.partial(
        pl.kernel, mesh=mesh,
        out_type=jax.ShapeDtypeStruct((tot, _TW), jnp.float32),
        scratch_types=[
            pltpu.VMEM((per_w,), jnp.int32),
            pltpu.VMEM((ch, _TW), jnp.float32),
            pltpu.VMEM((ch, _TW), jnp.float32),
            pltpu.VMEM((ch, _TW), jnp.float32),
            pltpu.SemaphoreType.DMA,
            pltpu.SemaphoreType.DMA,
            pltpu.SemaphoreType.DMA,
            pltpu.SemaphoreType.DMA,
            pltpu.SemaphoreType.DMA,
            pltpu.SemaphoreType.DMA,
        ],
    )
    def gather_kernel(table_hbm, idx_hbm, out_hbm, idx_v, rows_a, rows_b,
                      rows_c, gs_a, gs_b, gs_c, ws_a, ws_b, ws_c):
        wid = lax.axis_index("s") * 2 + lax.axis_index("c")
        base = wid * per_w
        pltpu.sync_copy(idx_hbm.at[pl.ds(base, per_w)], idx_v)
        bufs = (rows_a, rows_b, rows_c)
        gsems = (gs_a, gs_b, gs_c)
        wsems = (ws_a, ws_b, ws_c)
        gcp = [None] * 3
        wcp = [None] * 3
        nb = 3
        for c in range(n_ch):
            p = c % nb
            if c >= nb:
                wcp[p].wait()
            gcp[p] = pltpu.async_copy(
                table_hbm.at[idx_v.at[pl.ds(c * ch, ch)]],
                bufs[p], gsems[p])
            pp = (c - 1) % nb
            if c >= 1:
                gcp[pp].wait()
                wcp[pp] = pltpu.async_copy(
                    bufs[pp], out_hbm.at[pl.ds(base + (c - 1) * ch, ch)],
                    wsems[pp])
        pl_last = (n_ch - 1) % nb
        gcp[pl_last].wait()
        wcp[pl_last] = pltpu.async_copy(
            bufs[pl_last], out_hbm.at[pl.ds(base + (n_ch - 1) * ch, ch)],
            wsems[pl_last])
        for c in range(max(0, n_ch - nb), n_ch):
            wcp[c % nb].wait()

    return gather_kernel(table, idx_flat)


def _stage3_body(g_ref, xyz_ref, pre_ref, E_ref,
                 Aqb, cq, Akvb, ckv,
                 Wd1b, bd1, Wd2cb, bd2, Wg2b, bg2, W2, b2,
                 attn_ref, res_ref):
    g = g_ref[:]
    xq = xyz_ref[:]
    fq = pre_ref[:]
    pos = xq[:, None, :] - g[:, :, 0:_INF]
    pos2 = pos.reshape(_TM * _K, _INF)
    h = jnp.maximum(_bmm(pos2, Wd1b) + bd1[:], 0.0)
    hb = h.astype(_BF)
    hw = jnp.dot(hb, Wd2cb[:], preferred_element_type=jnp.float32)
    pe = hw[:, 0:_D] + bd2[:]
    a3 = hw[:, _D:]

    f2 = g[:, :, _TF:].reshape(_TM * _K, _TF)
    kv = _bmm(f2, Akvb) + ckv[:]
    kg2 = kv[:, 0:_D]
    v2 = kv[:, _D:]
    qg = _bmm(fq, Aqb) + cq[:]
    qg2 = jnp.broadcast_to(qg[:, None, :], (_TM, _K, _D))
    qg2 = qg2.reshape(_TM * _K, _D)

    l1 = jnp.maximum(qg2 - kg2 + a3, 0.0)
    logits = (_bmm(l1, Wg2b) + bg2[:]) * jnp.float32(1.0 / 16.0)

    lg3 = logits.reshape(_TM, _K, _D)
    m = jnp.max(lg3, axis=1, keepdims=True)
    e = jnp.exp(lg3 - m)
    s = jnp.sum(e, axis=1, keepdims=True)
    attn = e / s
    attn_ref[:] = attn

    pe3 = pe.reshape(_TM, _K, _D)
    v3 = v2.reshape(_TM, _K, _D)
    p2 = (attn * (v3 + pe3)).reshape(_TM * _K, _D)
    wsum = jnp.dot(E_ref[:], p2, preferred_element_type=jnp.float32)
    res_ref[:] = jnp.dot(wsum, W2[:]) + b2[:] + fq


def _stage3_body_alias(attn_in, res_in, *args):
    _stage3_body(*args)


def _stage3(batch, g3, xyzf, table, E, wp, prev):
    f32 = jnp.float32
    (Aqb, cq, Akvb, ckv, Wd1b, Wd2cb, Wg2b) = wp[0]
    (bd1, bd2, bg2, W2, b2) = wp[1]
    nt = _N // _TM
    grid = (nt,)
    row = lambda t: (batch * nt + t, 0)
    row3 = lambda t: (batch * nt + t, 0, 0)
    grow = lambda t: (t, 0, 0)
    row_f = lambda t: (batch * nt + t, 1)
    full = lambda t: (0, 0)
    in_specs = [
        pl.BlockSpec((_TM, _K, _TW), grow),
        pl.BlockSpec((_TM, _INF), row),
        pl.BlockSpec((_TM, _TF), row_f),
        pl.BlockSpec((_TM, _TM * _K), full),
        pl.BlockSpec((_TF, _D), full),
        pl.BlockSpec((1, _D), full),
        pl.BlockSpec((_TF, 2 * _D), full),
        pl.BlockSpec((1, 2 * _D), full),
        pl.BlockSpec((_INF, _D), full),
        pl.BlockSpec((1, _D), full),
        pl.BlockSpec((_D, 2 * _D), full),
        pl.BlockSpec((1, _D), full),
        pl.BlockSpec((_D, _D), full),
        pl.BlockSpec((1, _D), full),
        pl.BlockSpec((_D, _TF), full),
        pl.BlockSpec((1, _TF), full),
    ]
    out_specs = [
        pl.BlockSpec((_TM, _K, _D), row3),
        pl.BlockSpec((_TM, _TF), row),
    ]
    outs = (
        jax.ShapeDtypeStruct((_B * _N, _K, _D), f32),
        jax.ShapeDtypeStruct((_B * _N, _TF), f32),
    )
    operands = (g3, xyzf, table, E, Aqb, cq, Akvb, ckv,
                Wd1b, bd1.reshape(1, _D), Wd2cb, bd2.reshape(1, _D),
                Wg2b, bg2.reshape(1, _D), W2, b2.reshape(1, _TF))
    if prev is None:
        return pl.pallas_call(
            _stage3_body, grid=grid, in_specs=in_specs,
            out_specs=out_specs, out_shape=outs)(*operands)
    attn_prev, res_prev = prev
    in_specs = [pl.BlockSpec(memory_space=pl.ANY),
                pl.BlockSpec(memory_space=pl.ANY)] + in_specs
    return pl.pallas_call(
        _stage3_body_alias, grid=grid, in_specs=in_specs,
        out_specs=out_specs, out_shape=outs,
        input_output_aliases={0: 0, 1: 1},
    )(attn_prev, res_prev, *operands)


def kernel(xyz, W0a, b0a, W0b, b0b, W1, b1, W2, b2, Wd1, bd1, Wd2, bd2,
           Wg1, bg1, Wg2, bg2, Wq, Wk, Wv):
    assert xyz.shape == (_B, _N, _INF)
    Aqb, cq, Akvb, ckv, Wd2cb, Wd1b, Wg2b = _wprep(
        W1, b1, Wq, Wk, Wv, Wg1, Wd2, bd2, bg1, Wd1, Wg2)
    wp = ((Aqb, cq, Akvb, ckv, Wd1b, Wd2cb, Wg2b),
          (bd1, bd2, bg2, W2, b2))
    rows = jnp.arange(_TM, dtype=jnp.int32)[:, None]
    cols = jnp.arange(_TM * _K, dtype=jnp.int32)[None, :]
    E = (cols // _K == rows).astype(jnp.float32)
    xyzf = xyz.reshape(_B * _N, _INF)
    table = _feats(xyzf, W0a, b0a, W0b, b0b)
    prev = None
    for b in range(_B):
        idx = _topk(b, xyzf)
        g = _sc_gather(table, idx.reshape(_N * _K))
        prev = _stage3(b, g.reshape(_N, _K, _TW), xyzf, table, E, wp, prev)
    attn, res = prev[0], prev[1]
    return (res.reshape(_B, _N, _TF), attn.reshape(_B, _N, _K, _D))


# speedup vs baseline: 1.0689x; 1.0541x over previous
import functools

import jax
import jax.numpy as jnp
from jax import lax
from jax.experimental import pallas as pl
from jax.experimental.pallas import tpu as pltpu
from jax.experimental.pallas import tpu_sc as plsc

_HI = lax.Precision.HIGHEST

_B, _N, _INF, _TF, _D, _K = 2, 1024, 64, 128, 256, 16
_TQ = 256
_TM = 128
_TW = 256


def _wprep_body(W1, b1, Wq, Wk, Wv, Wg1, Wd2, bd2, bg1,
                Aq, cq, Akv, ckv, Wd2c):
    w1 = W1[:]
    g1 = Wg1[:]
    Aq[:] = jnp.dot(jnp.dot(w1, Wq[:], precision=_HI), g1, precision=_HI)
    Ak = jnp.dot(jnp.dot(w1, Wk[:], precision=_HI), g1, precision=_HI)
    Av = jnp.dot(w1, Wv[:], precision=_HI)
    Akv[:] = jnp.concatenate([Ak, Av], axis=1)
    b1v = b1[:]
    cq[:] = (jnp.dot(jnp.dot(b1v, Wq[:], precision=_HI), g1, precision=_HI)
             + jnp.dot(bd2[:], g1, precision=_HI) + bg1[:])
    ck = jnp.dot(jnp.dot(b1v, Wk[:], precision=_HI), g1, precision=_HI)
    cv = jnp.dot(b1v, Wv[:], precision=_HI)
    ckv[:] = jnp.concatenate([ck, cv], axis=1)
    Wd2g = jnp.dot(Wd2[:], g1, precision=_HI)
    Wd2c[:] = jnp.concatenate([Wd2[:], Wd2g], axis=1)


def _wprep(W1, b1, Wq, Wk, Wv, Wg1, Wd2, bd2, bg1):
    f32 = jnp.float32
    outs = (
        jax.ShapeDtypeStruct((_TF, _D), f32),
        jax.ShapeDtypeStruct((1, _D), f32),
        jax.ShapeDtypeStruct((_TF, 2 * _D), f32),
        jax.ShapeDtypeStruct((1, 2 * _D), f32),
        jax.ShapeDtypeStruct((_D, 2 * _D), f32),
    )
    return pl.pallas_call(_wprep_body, out_shape=outs)(
        W1, b1.reshape(1, _D), Wq, Wk, Wv, Wg1, Wd2,
        bd2.reshape(1, _D), bg1.reshape(1, _D))


def _topk_body(xq_ref, xf_ref, W0a, b0a, W0b, b0b, table_ref, idx_ref):
    xq = xq_ref[:]
    xf = xf_ref[:]

    f1 = jnp.maximum(jnp.dot(xq, W0a[:]) + b0a[:], 0.0)
    feats = jnp.dot(f1, W0b[:]) + b0b[:]
    table_ref[:] = jnp.concatenate(
        [xq, jnp.zeros((_TQ, _INF), jnp.float32), feats], axis=1)

    d = -2.0 * lax.dot_general(xq, xf, (((1,), (1,)), ((), ())))
    d = d + jnp.sum(xq * xq, axis=1, keepdims=True)
    d = d + jnp.sum(xf * xf, axis=1)[None, :]

    colf = lax.broadcasted_iota(jnp.int32, (_TQ, _N), 1).astype(jnp.float32)
    big = jnp.float32(3.0e38)
    vals = d
    sels = []
    for _ in range(_K):
        m = jnp.min(vals, axis=1, keepdims=True)
        cand = jnp.where(vals <= m, colf, jnp.float32(_N))
        sel = jnp.min(cand, axis=1, keepdims=True)
        sels.append(sel)
        vals = jnp.where(colf == sel, big, vals)
    idx_ref[:] = jnp.concatenate(sels, axis=1).astype(jnp.int32)


def _topk(batch, xyzf, W0a, b0a, W0b, b0b):
    nt = _N // _TQ
    grid = (nt,)
    full = lambda t: (0, 0)
    return pl.pallas_call(
        functools.partial(_topk_body),
        grid=grid,
        in_specs=[
            pl.BlockSpec((_TQ, _INF), lambda t: (batch * nt + t, 0)),
            pl.BlockSpec((_N, _INF), lambda t: (batch, 0)),
            pl.BlockSpec((_INF, _TF), full),
            pl.BlockSpec((1, _TF), full),
            pl.BlockSpec((_TF, _TF), full),
            pl.BlockSpec((1, _TF), full),
        ],
        out_specs=[
            pl.BlockSpec((_TQ, _TW), lambda t: (t, 0)),
            pl.BlockSpec((_TQ, _K), lambda t: (t, 0)),
        ],
        out_shape=(
            jax.ShapeDtypeStruct((_N, _TW), jnp.float32),
            jax.ShapeDtypeStruct((_N, _K), jnp.int32),
        ),
    )(xyzf, xyzf, W0a, b0a.reshape(1, _TF), W0b, b0b.reshape(1, _TF))


def _sc_gather(table, idx_flat):
    tot = idx_flat.shape[0]
    nw = 32
    per_w = tot // nw
    ch = 128
    n_ch = per_w // ch

    mesh = plsc.VectorSubcoreMesh(core_axis_name="c", subcore_axis_name="s")

    @functools.partial(
        pl.kernel, mesh=mesh,
        out_type=jax.ShapeDtypeStruct((tot, _TW), jnp.float32),
        scratch_types=[
            pltpu.VMEM((per_w,), jnp.int32),
            pltpu.VMEM((ch, _TW), jnp.float32),
            pltpu.VMEM((ch, _TW), jnp.float32),
            pltpu.VMEM((ch, _TW), jnp.float32),
            pltpu.SemaphoreType.DMA,
            pltpu.SemaphoreType.DMA,
            pltpu.SemaphoreType.DMA,
            pltpu.SemaphoreType.DMA,
            pltpu.SemaphoreType.DMA,
            pltpu.SemaphoreType.DMA,
        ],
    )
    def gather_kernel(table_hbm, idx_hbm, out_hbm, idx_v, rows_a, rows_b,
                      rows_c, gs_a, gs_b, gs_c, ws_a, ws_b, ws_c):
        wid = lax.axis_index("s") * 2 + lax.axis_index("c")
        base = wid * per_w
        pltpu.sync_copy(idx_hbm.at[pl.ds(base, per_w)], idx_v)
        bufs = (rows_a, rows_b, rows_c)
        gsems = (gs_a, gs_b, gs_c)
        wsems = (ws_a, ws_b, ws_c)
        gcp = [None] * 3
        wcp = [None] * 3
        nb = 3
        for c in range(n_ch):
            p = c % nb
            if c >= nb:
                wcp[p].wait()
            gcp[p] = pltpu.async_copy(
                table_hbm.at[idx_v.at[pl.ds(c * ch, ch)]],
                bufs[p], gsems[p])
            pp = (c - 1) % nb
            if c >= 1:
                gcp[pp].wait()
                wcp[pp] = pltpu.async_copy(
                    bufs[pp], out_hbm.at[pl.ds(base + (c - 1) * ch, ch)],
                    wsems[pp])
        pl_last = (n_ch - 1) % nb
        gcp[pl_last].wait()
        wcp[pl_last] = pltpu.async_copy(
            bufs[pl_last], out_hbm.at[pl.ds(base + (n_ch - 1) * ch, ch)],
            wsems[pl_last])
        for c in range(max(0, n_ch - nb), n_ch):
            wcp[c % nb].wait()

    return gather_kernel(table, idx_flat)


def _stage3_body(g_ref, xyz_ref, pre_ref,
                 Aq, cq, Akv, ckv,
                 Wd1, bd1, Wd2c, bd2, Wg2, bg2, W2, b2,
                 attn_ref, res_ref):
    g = g_ref[:]
    xq = xyz_ref[:]
    fq = pre_ref[:]
    pos = xq[:, None, :] - g[:, :, 0:_INF]
    pos2 = pos.reshape(_TM * _K, _INF)
    h = jnp.maximum(jnp.dot(pos2, Wd1[:]) + bd1[:], 0.0)
    hw = jnp.dot(h, Wd2c[:])
    pe = hw[:, 0:_D] + bd2[:]
    a3 = hw[:, _D:]

    f2 = g[:, :, _TF:].reshape(_TM * _K, _TF)
    kv = jnp.dot(f2, Akv[:]) + ckv[:]
    kg2 = kv[:, 0:_D]
    v2 = kv[:, _D:]
    qg = jnp.dot(fq, Aq[:]) + cq[:]
    qg2 = jnp.broadcast_to(qg[:, None, :], (_TM, _K, _D))
    qg2 = qg2.reshape(_TM * _K, _D)

    l1 = jnp.maximum(qg2 - kg2 + a3, 0.0)
    logits = (jnp.dot(l1, Wg2[:]) + bg2[:]) * jnp.float32(1.0 / 16.0)

    lg3 = logits.reshape(_TM, _K, _D)
    m = jnp.max(lg3, axis=1, keepdims=True)
    e = jnp.exp(lg3 - m)
    s = jnp.sum(e, axis=1, keepdims=True)
    attn = e / s
    attn_ref[:] = attn

    pe3 = pe.reshape(_TM, _K, _D)
    v3 = v2.reshape(_TM, _K, _D)
    wsum = jnp.sum(attn * (v3 + pe3), axis=1)
    res_ref[:] = jnp.dot(wsum, W2[:]) + b2[:] + fq


def _stage3_body_alias(attn_in, res_in, *args):
    _stage3_body(*args)


def _stage3(batch, g3, xyzf, table_b, wp, prev):
    f32 = jnp.float32
    (Aq, cq, Akv, ckv, Wd2c) = wp[0]
    (Wd1, bd1, bd2, Wg2, bg2, W2, b2) = wp[1]
    nt = _N // _TM
    grid = (nt,)
    row = lambda t: (batch * nt + t, 0)
    row3 = lambda t: (batch * nt + t, 0, 0)
    grow = lambda t: (t, 0, 0)
    trow_f = lambda t: (t, 1)
    full = lambda t: (0, 0)
    in_specs = [
        pl.BlockSpec((_TM, _K, _TW), grow),
        pl.BlockSpec((_TM, _INF), row),
        pl.BlockSpec((_TM, _TF), trow_f),
        pl.BlockSpec((_TF, _D), full),
        pl.BlockSpec((1, _D), full),
        pl.BlockSpec((_TF, 2 * _D), full),
        pl.BlockSpec((1, 2 * _D), full),
        pl.BlockSpec((_INF, _D), full),
        pl.BlockSpec((1, _D), full),
        pl.BlockSpec((_D, 2 * _D), full),
        pl.BlockSpec((1, _D), full),
        pl.BlockSpec((_D, _D), full),
        pl.BlockSpec((1, _D), full),
        pl.BlockSpec((_D, _TF), full),
        pl.BlockSpec((1, _TF), full),
    ]
    out_specs = [
        pl.BlockSpec((_TM, _K, _D), row3),
        pl.BlockSpec((_TM, _TF), row),
    ]
    outs = (
        jax.ShapeDtypeStruct((_B * _N, _K, _D), f32),
        jax.ShapeDtypeStruct((_B * _N, _TF), f32),
    )
    operands = (g3, xyzf, table_b, Aq, cq, Akv, ckv,
                Wd1, bd1.reshape(1, _D), Wd2c, bd2.reshape(1, _D),
                Wg2, bg2.reshape(1, _D), W2, b2.reshape(1, _TF))
    if prev is None:
        return pl.pallas_call(
            _stage3_body, grid=grid, in_specs=in_specs,
            out_specs=out_specs, out_shape=outs)(*operands)
    attn_prev, res_prev = prev
    in_specs = [pl.BlockSpec(memory_space=pl.ANY),
                pl.BlockSpec(memory_space=pl.ANY)] + in_specs
    return pl.pallas_call(
        _stage3_body_alias, grid=grid, in_specs=in_specs,
        out_specs=out_specs, out_shape=outs,
        input_output_aliases={0: 0, 1: 1},
    )(attn_prev, res_prev, *operands)


def kernel(xyz, W0a, b0a, W0b, b0b, W1, b1, W2, b2, Wd1, bd1, Wd2, bd2,
           Wg1, bg1, Wg2, bg2, Wq, Wk, Wv):
    assert xyz.shape == (_B, _N, _INF)
    Aq, cq, Akv, ckv, Wd2c = _wprep(W1, b1, Wq, Wk, Wv, Wg1, Wd2, bd2, bg1)
    wp = ((Aq, cq, Akv, ckv, Wd2c), (Wd1, bd1, bd2, Wg2, bg2, W2, b2))
    xyzf = xyz.reshape(_B * _N, _INF)
    prev = None
    for b in range(_B):
        table_b, idx = _topk(b, xyzf, W0a, b0a, W0b, b0b)
        g = _sc_gather(table_b, idx.reshape(_N * _K))
        prev = _stage3(b, g.reshape(_N, _K, _TW), xyzf, table_b, wp, prev)
    attn, res = prev[0], prev[1]
    return (res.reshape(_B, _N, _TF), attn.reshape(_B, _N, _K, _D))


# speedup vs baseline: 1.0797x; 1.0101x over previous
import functools

import jax
import jax.numpy as jnp
from jax import lax
from jax.experimental import pallas as pl
from jax.experimental.pallas import tpu as pltpu
from jax.experimental.pallas import tpu_sc as plsc

_HI = lax.Precision.HIGHEST

_B, _N, _INF, _TF, _D, _K = 2, 1024, 64, 128, 256, 16
_TQ = 512
_TM = 128
_TW = 256


def _wprep_body(W1, b1, Wq, Wk, Wv, Wg1, Wd2, bd2, bg1,
                Aq, cq, Akv, ckv, Wd2c):
    w1 = W1[:]
    g1 = Wg1[:]
    Aq[:] = jnp.dot(jnp.dot(w1, Wq[:], precision=_HI), g1, precision=_HI)
    Ak = jnp.dot(jnp.dot(w1, Wk[:], precision=_HI), g1, precision=_HI)
    Av = jnp.dot(w1, Wv[:], precision=_HI)
    Akv[:] = jnp.concatenate([Ak, Av], axis=1)
    b1v = b1[:]
    cq[:] = (jnp.dot(jnp.dot(b1v, Wq[:], precision=_HI), g1, precision=_HI)
             + jnp.dot(bd2[:], g1, precision=_HI) + bg1[:])
    ck = jnp.dot(jnp.dot(b1v, Wk[:], precision=_HI), g1, precision=_HI)
    cv = jnp.dot(b1v, Wv[:], precision=_HI)
    ckv[:] = jnp.concatenate([ck, cv], axis=1)
    Wd2g = jnp.dot(Wd2[:], g1, precision=_HI)
    Wd2c[:] = jnp.concatenate([Wd2[:], Wd2g], axis=1)


def _wprep(W1, b1, Wq, Wk, Wv, Wg1, Wd2, bd2, bg1):
    f32 = jnp.float32
    outs = (
        jax.ShapeDtypeStruct((_TF, _D), f32),
        jax.ShapeDtypeStruct((1, _D), f32),
        jax.ShapeDtypeStruct((_TF, 2 * _D), f32),
        jax.ShapeDtypeStruct((1, 2 * _D), f32),
        jax.ShapeDtypeStruct((_D, 2 * _D), f32),
    )
    return pl.pallas_call(_wprep_body, out_shape=outs)(
        W1, b1.reshape(1, _D), Wq, Wk, Wv, Wg1, Wd2,
        bd2.reshape(1, _D), bg1.reshape(1, _D))


def _topk_body(xq_ref, xf_ref, W0a, b0a, W0b, b0b, table_ref, idx_ref):
    xq = xq_ref[:]
    xf = xf_ref[:]

    f1 = jnp.maximum(jnp.dot(xq, W0a[:]) + b0a[:], 0.0)
    feats = jnp.dot(f1, W0b[:]) + b0b[:]
    table_ref[:] = jnp.concatenate(
        [xq, jnp.zeros((_TQ, _INF), jnp.float32), feats], axis=1)

    d = -2.0 * lax.dot_general(xq, xf, (((1,), (1,)), ((), ())))
    d = d + jnp.sum(xq * xq, axis=1, keepdims=True)
    d = d + jnp.sum(xf * xf, axis=1)[None, :]

    colf = lax.broadcasted_iota(jnp.int32, (_TQ, _N), 1).astype(jnp.float32)
    big = jnp.float32(3.0e38)
    vals = d
    sels = []
    for _ in range(_K):
        m = jnp.min(vals, axis=1, keepdims=True)
        cand = jnp.where(vals <= m, colf, jnp.float32(_N))
        sel = jnp.min(cand, axis=1, keepdims=True)
        sels.append(sel)
        vals = jnp.where(colf == sel, big, vals)
    idx_ref[:] = jnp.concatenate(sels, axis=1).astype(jnp.int32)


def _topk(batch, xyzf, W0a, b0a, W0b, b0b):
    nt = _N // _TQ
    grid = (nt,)
    full = lambda t: (0, 0)
    return pl.pallas_call(
        functools.partial(_topk_body),
        grid=grid,
        in_specs=[
            pl.BlockSpec((_TQ, _INF), lambda t: (batch * nt + t, 0)),
            pl.BlockSpec((_N, _INF), lambda t: (batch, 0)),
            pl.BlockSpec((_INF, _TF), full),
            pl.BlockSpec((1, _TF), full),
            pl.BlockSpec((_TF, _TF), full),
            pl.BlockSpec((1, _TF), full),
        ],
        out_specs=[
            pl.BlockSpec((_TQ, _TW), lambda t: (t, 0)),
            pl.BlockSpec((_TQ, _K), lambda t: (t, 0)),
        ],
        out_shape=(
            jax.ShapeDtypeStruct((_N, _TW), jnp.float32),
            jax.ShapeDtypeStruct((_N, _K), jnp.int32),
        ),
    )(xyzf, xyzf, W0a, b0a.reshape(1, _TF), W0b, b0b.reshape(1, _TF))


def _sc_gather(table, idx_flat):
    tot = idx_flat.shape[0]
    nw = 32
    per_w = tot // nw
    ch = 128
    n_ch = per_w // ch

    mesh = plsc.VectorSubcoreMesh(core_axis_name="c", subcore_axis_name="s")

    @functools.partial(
        pl.kernel, mesh=mesh,
        out_type=jax.ShapeDtypeStruct((tot, _TW), jnp.float32),
        scratch_types=[
            pltpu.VMEM((per_w,), jnp.int32),
            pltpu.VMEM((ch, _TW), jnp.float32),
            pltpu.VMEM((ch, _TW), jnp.float32),
            pltpu.VMEM((ch, _TW), jnp.float32),
            pltpu.SemaphoreType.DMA,
            pltpu.SemaphoreType.DMA,
            pltpu.SemaphoreType.DMA,
            pltpu.SemaphoreType.DMA,
            pltpu.SemaphoreType.DMA,
            pltpu.SemaphoreType.DMA,
        ],
    )
    def gather_kernel(table_hbm, idx_hbm, out_hbm, idx_v, rows_a, rows_b,
                      rows_c, gs_a, gs_b, gs_c, ws_a, ws_b, ws_c):
        wid = lax.axis_index("s") * 2 + lax.axis_index("c")
        base = wid * per_w
        pltpu.sync_copy(idx_hbm.at[pl.ds(base, per_w)], idx_v)
        bufs = (rows_a, rows_b, rows_c)
        gsems = (gs_a, gs_b, gs_c)
        wsems = (ws_a, ws_b, ws_c)
        gcp = [None] * 3
        wcp = [None] * 3
        nb = 3
        for c in range(n_ch):
            p = c % nb
            if c >= nb:
                wcp[p].wait()
            gcp[p] = pltpu.async_copy(
                table_hbm.at[idx_v.at[pl.ds(c * ch, ch)]],
                bufs[p], gsems[p])
            pp = (c - 1) % nb
            if c >= 1:
                gcp[pp].wait()
                wcp[pp] = pltpu.async_copy(
                    bufs[pp], out_hbm.at[pl.ds(base + (c - 1) * ch, ch)],
                    wsems[pp])
        pl_last = (n_ch - 1) % nb
        gcp[pl_last].wait()
        wcp[pl_last] = pltpu.async_copy(
            bufs[pl_last], out_hbm.at[pl.ds(base + (n_ch - 1) * ch, ch)],
            wsems[pl_last])
        for c in range(max(0, n_ch - nb), n_ch):
            wcp[c % nb].wait()

    return gather_kernel(table, idx_flat)


def _stage3_body(g_ref, xyz_ref, pre_ref,
                 Aq, cq, Akv, ckv,
                 Wd1, bd1, Wd2c, bd2, Wg2, bg2, W2, b2,
                 attn_ref, res_ref):
    g = g_ref[:]
    xq = xyz_ref[:]
    fq = pre_ref[:]
    pos = xq[:, None, :] - g[:, :, 0:_INF]
    pos2 = pos.reshape(_TM * _K, _INF)
    h = jnp.maximum(jnp.dot(pos2, Wd1[:]) + bd1[:], 0.0)
    hw = jnp.dot(h, Wd2c[:])
    pe = hw[:, 0:_D] + bd2[:]
    a3 = hw[:, _D:]

    f2 = g[:, :, _TF:].reshape(_TM * _K, _TF)
    kv = jnp.dot(f2, Akv[:]) + ckv[:]
    kg2 = kv[:, 0:_D]
    v2 = kv[:, _D:]
    qg = jnp.dot(fq, Aq[:]) + cq[:]
    qg2 = jnp.broadcast_to(qg[:, None, :], (_TM, _K, _D))
    qg2 = qg2.reshape(_TM * _K, _D)

    l1 = jnp.maximum(qg2 - kg2 + a3, 0.0)
    logits = (jnp.dot(l1, Wg2[:]) + bg2[:]) * jnp.float32(1.0 / 16.0)

    lg3 = logits.reshape(_TM, _K, _D)
    m = jnp.max(lg3, axis=1, keepdims=True)
    e = jnp.exp(lg3 - m)
    s = jnp.sum(e, axis=1, keepdims=True)
    attn = e / s
    attn_ref[:] = attn

    pe3 = pe.reshape(_TM, _K, _D)
    v3 = v2.reshape(_TM, _K, _D)
    wsum = jnp.sum(attn * (v3 + pe3), axis=1)
    res_ref[:] = jnp.dot(wsum, W2[:]) + b2[:] + fq


def _stage3_body_alias(attn_in, res_in, *args):
    _stage3_body(*args)


def _stage3(batch, g3, xyzf, table_b, wp, prev):
    f32 = jnp.float32
    (Aq, cq, Akv, ckv, Wd2c) = wp[0]
    (Wd1, bd1, bd2, Wg2, bg2, W2, b2) = wp[1]
    nt = _N // _TM
    grid = (nt,)
    row = lambda t: (batch * nt + t, 0)
    row3 = lambda t: (batch * nt + t, 0, 0)
    grow = lambda t: (t, 0, 0)
    trow_f = lambda t: (t, 1)
    full = lambda t: (0, 0)
    in_specs = [
        pl.BlockSpec((_TM, _K, _TW), grow),
        pl.BlockSpec((_TM, _INF), row),
        pl.BlockSpec((_TM, _TF), trow_f),
        pl.BlockSpec((_TF, _D), full),
        pl.BlockSpec((1, _D), full),
        pl.BlockSpec((_TF, 2 * _D), full),
        pl.BlockSpec((1, 2 * _D), full),
        pl.BlockSpec((_INF, _D), full),
        pl.BlockSpec((1, _D), full),
        pl.BlockSpec((_D, 2 * _D), full),
        pl.BlockSpec((1, _D), full),
        pl.BlockSpec((_D, _D), full),
        pl.BlockSpec((1, _D), full),
        pl.BlockSpec((_D, _TF), full),
        pl.BlockSpec((1, _TF), full),
    ]
    out_specs = [
        pl.BlockSpec((_TM, _K, _D), row3),
        pl.BlockSpec((_TM, _TF), row),
    ]
    outs = (
        jax.ShapeDtypeStruct((_B * _N, _K, _D), f32),
        jax.ShapeDtypeStruct((_B * _N, _TF), f32),
    )
    operands = (g3, xyzf, table_b, Aq, cq, Akv, ckv,
                Wd1, bd1.reshape(1, _D), Wd2c, bd2.reshape(1, _D),
                Wg2, bg2.reshape(1, _D), W2, b2.reshape(1, _TF))
    if prev is None:
        return pl.pallas_call(
            _stage3_body, grid=grid, in_specs=in_specs,
            out_specs=out_specs, out_shape=outs)(*operands)
    attn_prev, res_prev = prev
    in_specs = [pl.BlockSpec(memory_space=pl.ANY),
                pl.BlockSpec(memory_space=pl.ANY)] + in_specs
    return pl.pallas_call(
        _stage3_body_alias, grid=grid, in_specs=in_specs,
        out_specs=out_specs, out_shape=outs,
        input_output_aliases={0: 0, 1: 1},
    )(attn_prev, res_prev, *operands)


def kernel(xyz, W0a, b0a, W0b, b0b, W1, b1, W2, b2, Wd1, bd1, Wd2, bd2,
           Wg1, bg1, Wg2, bg2, Wq, Wk, Wv):
    assert xyz.shape == (_B, _N, _INF)
    Aq, cq, Akv, ckv, Wd2c = _wprep(W1, b1, Wq, Wk, Wv, Wg1, Wd2, bd2, bg1)
    wp = ((Aq, cq, Akv, ckv, Wd2c), (Wd1, bd1, bd2, Wg2, bg2, W2, b2))
    xyzf = xyz.reshape(_B * _N, _INF)
    prev = None
    for b in range(_B):
        table_b, idx = _topk(b, xyzf, W0a, b0a, W0b, b0b)
        g = _sc_gather(table_b, idx.reshape(_N * _K))
        prev = _stage3(b, g.reshape(_N, _K, _TW), xyzf, table_b, wp, prev)
    attn, res = prev[0], prev[1]
    return (res.reshape(_B, _N, _TF), attn.reshape(_B, _N, _K, _D))
